# baseline (speedup 1.0000x reference)
import functools

import jax
import jax.numpy as jnp
from jax.experimental import pallas as pl


def _combine_kernel(idx_ref, w_ref, v_ref, out_ref, eff_ref):
    i = pl.program_id(0)
    idxt = idx_ref[:]
    wt = w_ref[:]
    k, bt = idxt.shape
    num_experts, d_model = v_ref.shape

    total = jnp.sum(wt, axis=0, keepdims=True)
    recip = jnp.where(total > 0, 1.0 / total, 1.0)
    wn = wt * recip

    e_iota = jax.lax.broadcasted_iota(jnp.int32, (num_experts, bt), 0)
    st = jnp.zeros((num_experts, bt), dtype=jnp.float32)
    for kk in range(k):
        st = st + jnp.where(idxt[kk:kk + 1, :] == e_iota, wn[kk:kk + 1, :], 0.0)

    out = jax.lax.dot_general(
        st, v_ref[:], dimension_numbers=(((0,), (0,)), ((), ())),
        preferred_element_type=jnp.float32)
    out_ref[:] = out

    sq = out * out
    ones_col = jnp.ones((d_model, 1), dtype=jnp.float32)
    ssq = jax.lax.dot_general(
        sq, ones_col, dimension_numbers=(((1,), (0,)), ((), ())),
        preferred_element_type=jnp.float32)
    norms = jnp.sqrt(ssq)
    partial = jnp.sum(norms).reshape(1, 1)

    @pl.when(i == 0)
    def _init():
        eff_ref[:] = partial

    @pl.when(i > 0)
    def _acc():
        eff_ref[:] = eff_ref[:] + partial


@jax.jit
def kernel(expert_indices, expert_weights, vertices):
    tokens, k = expert_indices.shape
    num_experts, d_model = vertices.shape
    bt = 4096
    grid = (tokens // bt,)

    idx_t = expert_indices.astype(jnp.int32).T
    w_t = expert_weights.T

    out, eff = pl.pallas_call(
        _combine_kernel,
        grid=grid,
        in_specs=[
            pl.BlockSpec((k, bt), lambda i: (0, i)),
            pl.BlockSpec((k, bt), lambda i: (0, i)),
            pl.BlockSpec((num_experts, d_model), lambda i: (0, 0)),
        ],
        out_specs=[
            pl.BlockSpec((bt, d_model), lambda i: (i, 0)),
            pl.BlockSpec((1, 1), lambda i: (0, 0)),
        ],
        out_shape=[
            jax.ShapeDtypeStruct((tokens, d_model), jnp.float32),
            jax.ShapeDtypeStruct((1, 1), jnp.float32),
        ],
    )(idx_t, w_t, vertices)

    efficiency = (eff[0, 0] / tokens).astype(jnp.float32)
    return (out, efficiency)


# speedup vs baseline: 1.0203x; 1.0203x over previous
import functools

import jax
import jax.numpy as jnp
from jax.experimental import pallas as pl


def _combine_kernel(idx_ref, w_ref, v_ref, out_ref, eff_ref):
    i = pl.program_id(0)
    idxt = idx_ref[:]
    wt = w_ref[:]
    k, bt = idxt.shape
    num_experts, d_model = v_ref.shape

    total = jnp.sum(wt, axis=0, keepdims=True)
    recip = jnp.where(total > 0, 1.0 / total, 1.0)
    wn = wt * recip

    e_iota = jax.lax.broadcasted_iota(jnp.int32, (num_experts, bt), 0)
    st = jnp.zeros((num_experts, bt), dtype=jnp.float32)
    for kk in range(k):
        st = st + jnp.where(idxt[kk:kk + 1, :] == e_iota, wn[kk:kk + 1, :], 0.0)

    out = jax.lax.dot_general(
        st, v_ref[:], dimension_numbers=(((0,), (0,)), ((), ())),
        preferred_element_type=jnp.float32)
    out_ref[:] = out

    sq = out * out
    ones_col = jnp.ones((d_model, 1), dtype=jnp.float32)
    ssq = jax.lax.dot_general(
        sq, ones_col, dimension_numbers=(((1,), (0,)), ((), ())),
        preferred_element_type=jnp.float32)
    norms = jnp.sqrt(ssq)
    partial = jnp.sum(norms).reshape(1, 1)

    @pl.when(i == 0)
    def _init():
        eff_ref[:] = partial

    @pl.when(i > 0)
    def _acc():
        eff_ref[:] = eff_ref[:] + partial


@jax.jit
def kernel(expert_indices, expert_weights, vertices):
    tokens, k = expert_indices.shape
    num_experts, d_model = vertices.shape
    bt = 2048
    grid = (tokens // bt,)

    idx_t = expert_indices.astype(jnp.int32).T
    w_t = expert_weights.T

    out, eff = pl.pallas_call(
        _combine_kernel,
        grid=grid,
        in_specs=[
            pl.BlockSpec((k, bt), lambda i: (0, i)),
            pl.BlockSpec((k, bt), lambda i: (0, i)),
            pl.BlockSpec((num_experts, d_model), lambda i: (0, 0)),
        ],
        out_specs=[
            pl.BlockSpec((bt, d_model), lambda i: (i, 0)),
            pl.BlockSpec((1, 1), lambda i: (0, 0)),
        ],
        out_shape=[
            jax.ShapeDtypeStruct((tokens, d_model), jnp.float32),
            jax.ShapeDtypeStruct((1, 1), jnp.float32),
        ],
    )(idx_t, w_t, vertices)

    efficiency = (eff[0, 0] / tokens).astype(jnp.float32)
    return (out, efficiency)
